# baseline (speedup 1.0000x reference)
import functools

import jax
import jax.numpy as jnp
from jax import lax
from jax.experimental import pallas as pl
from jax.experimental.pallas import tpu as pltpu
from jax.experimental.pallas import tpu_sc as plsc

_L = 10
_B = 4096
_N = 128
_NSC = 2
_NSUB = 16
_RPT = _B // _NSUB
_CPS = _N // _NSC
_NCH = _N // 16


def _reduce_body(spikes_ref, out_ref):
    out_ref[...] = jnp.sum(spikes_ref[...], axis=0)


def _time_reduce(st):
    T, B, N = st.shape
    Bb = 512
    G = B // Bb
    return pl.pallas_call(
        _reduce_body,
        grid=(G,),
        in_specs=[pl.BlockSpec((T, Bb, N), lambda i: (0, i, 0))],
        out_specs=pl.BlockSpec((Bb, N), lambda i: (i, 0)),
        out_shape=jax.ShapeDtypeStruct((B, N), jnp.float32),
        compiler_params=pltpu.CompilerParams(
            dimension_semantics=("arbitrary",)),
    )(st)


_AW = 16 * _N


def _sc_body(ssum_hbm, labels_hbm, assign_hbm, rates_hbm,
             chunk_v, lab_v, acc_v, comb_v, ratesbuf_v, abuf_v, parts_sh):
    c = lax.axis_index("c")
    s = lax.axis_index("s")
    row0 = s * _RPT
    col0 = c * _CPS
    iota16 = lax.iota(jnp.int32, 16)
    zero16 = jnp.zeros((16,), jnp.float32)

    for q in range(_AW // 16):
        acc_v[pl.ds(q * 16, 16)] = zero16

    pltpu.sync_copy(ssum_hbm.at[pl.ds(row0, _RPT)], chunk_v)
    pltpu.sync_copy(labels_hbm.at[pl.ds(row0, _RPT)], lab_v)

    def group_step(g, cnts):
        labv = lab_v[pl.ds(g * 16, 16)]
        for k in range(16):
            lab = labv[k]
            base = lab * _N
            row = g * 16 + k
            for j in range(_NCH):
                val = chunk_v[row, pl.ds(j * 16, 16)]
                plsc.addupdate_scatter(
                    acc_v, [iota16 + (base + j * 16)], val)
            cnts = cnts + jnp.where(iota16 == lab, 1.0, 0.0)
        return cnts

    cnts = lax.fori_loop(0, _RPT // 16, group_step, zero16)
    acc_v[pl.ds(_L * _N, 16)] = cnts

    pltpu.sync_copy(acc_v, parts_sh.at[s])
    plsc.subcore_barrier()

    @pl.when(s < _CPS // 16)
    def _fin():
        pltpu.sync_copy(parts_sh, comb_v)
        n0 = col0 + s * 16
        cv = comb_v[0, pl.ds(_L * _N, 16)]
        for k in range(1, _NSUB):
            cv = cv + comb_v[k, pl.ds(_L * _N, 16)]
        best = jnp.full((16,), -jnp.inf, jnp.float32)
        bidx = jnp.zeros((16,), jnp.int32)
        for l in range(_L):
            idx = iota16 + (l * _N) + n0
            tot = plsc.load_gather(comb_v, [jnp.zeros((16,), jnp.int32), idx])
            for k in range(1, _NSUB):
                tot = tot + plsc.load_gather(
                    comb_v, [jnp.full((16,), k), idx])
            cl = jnp.full((16,), cv[l])
            m = jnp.where(cl > 0.0, tot / jnp.maximum(cl, 1.0), 0.0)
            plsc.store_scatter(ratesbuf_v, [iota16 * _L + l], m)
            upd = m > best
            best = jnp.where(upd, m, best)
            bidx = jnp.where(upd, l, bidx)
        abuf_v[...] = bidx
        pltpu.sync_copy(abuf_v, assign_hbm.at[pl.ds(n0, 16)])
        pltpu.sync_copy(ratesbuf_v, rates_hbm.at[pl.ds(n0 * _L, 16 * _L)])


_sc_segment = functools.partial(
    pl.kernel,
    mesh=plsc.VectorSubcoreMesh(core_axis_name="c", subcore_axis_name="s"),
    compiler_params=pltpu.CompilerParams(needs_layout_passes=False),
    out_type=[
        jax.ShapeDtypeStruct((_N,), jnp.int32),
        jax.ShapeDtypeStruct((_N * _L,), jnp.float32),
    ],
    scratch_types=[
        pltpu.VMEM((_RPT, _N), jnp.float32),
        pltpu.VMEM((_RPT,), jnp.int32),
        pltpu.VMEM((_AW,), jnp.float32),
        pltpu.VMEM((_NSUB, _AW), jnp.float32),
        pltpu.VMEM((16 * _L,), jnp.float32),
        pltpu.VMEM((16,), jnp.int32),
        pltpu.VMEM_SHARED((_NSUB, _AW), jnp.float32),
    ],
)(_sc_body)


def kernel(spikes, labels):
    B, T, N = spikes.shape
    st = jnp.transpose(spikes, (1, 0, 2))
    ssum = _time_reduce(st)
    assign, rates_flat = _sc_segment(ssum, labels.astype(jnp.int32))
    return assign, rates_flat.reshape(N, _L)


# speedup vs baseline: 1.1392x; 1.1392x over previous
import functools

import jax
import jax.numpy as jnp
from jax import lax
from jax.experimental import pallas as pl
from jax.experimental.pallas import tpu as pltpu
from jax.experimental.pallas import tpu_sc as plsc

_L = 10
_B = 4096
_N = 128
_NSC = 1
_NSUB = 16
_RPT = _B // _NSUB
_CPS = _N // _NSC
_NCH = _N // 16


def _reduce_body(spikes_ref, out_ref):
    out_ref[...] = jnp.sum(spikes_ref[...], axis=0)


def _time_reduce(st):
    T, B, N = st.shape
    Bb = 512
    G = B // Bb
    return pl.pallas_call(
        _reduce_body,
        grid=(G,),
        in_specs=[pl.BlockSpec((T, Bb, N), lambda i: (0, i, 0))],
        out_specs=pl.BlockSpec((Bb, N), lambda i: (i, 0)),
        out_shape=jax.ShapeDtypeStruct((B, N), jnp.float32),
        compiler_params=pltpu.CompilerParams(
            dimension_semantics=("arbitrary",)),
    )(st)


def _sc_body(ssum_hbm, labels_hbm, assign_hbm, rates_hbm,
             chunk_v, lab_v, idxadj_v, zbuf_v, locfull_v, crow_v, cbuf_v,
             ratesbuf_v, abuf_v, sem, lsem, acc_sh, cnt_sh):
    c = lax.axis_index("c")
    s = lax.axis_index("s")
    row0 = s * _RPT
    col0 = c * _CPS
    iota16 = lax.iota(jnp.int32, 16)
    zero16 = jnp.zeros((16,), jnp.float32)

    for r in range(16):
        for j in range(_NCH):
            zbuf_v[r, pl.ds(j * 16, 16)] = zero16

    @pl.when(s == 0)
    def _zero_shared():
        pltpu.sync_copy(zbuf_v, acc_sh.at[pl.ds(c * 16, 16)])

    for h in range(2):
        pltpu.sync_copy(ssum_hbm.at[pl.ds(row0 + h * 128, 128)],
                        chunk_v.at[h])
    pltpu.sync_copy(labels_hbm.at[pl.ds(row0, _RPT)], lab_v)

    cnts = zero16
    for g in range(_RPT // 32):
        for h in range(2):
            labv = lab_v[pl.ds(h * 128 + g * 16, 16)]
            idxadj_v[h, pl.ds(g * 16, 16)] = labv + c * 16
            for k in range(16):
                cnts = cnts + jnp.where(iota16 == labv[k], 1.0, 0.0)

    for j in range(_NCH):
        crow_v[0, pl.ds(j * 16, 16)] = zero16
    crow_v[0, pl.ds(0, 16)] = cnts
    pltpu.sync_copy(crow_v, cnt_sh.at[pl.ds(s, 1)])

    plsc.subcore_barrier()
    for h in range(2):
        pltpu.sync_copy(chunk_v.at[h], acc_sh.at[idxadj_v.at[h]], add=True)
    plsc.subcore_barrier()

    @pl.when(s < _CPS // 16)
    def _fin():
        pltpu.sync_copy(acc_sh.at[pl.ds(c * 16, 16)], locfull_v)
        pltpu.sync_copy(cnt_sh, cbuf_v)
        n0 = col0 + s * 16
        cv = cbuf_v[0, pl.ds(0, 16)]
        for k in range(1, _NSUB):
            cv = cv + cbuf_v[k, pl.ds(0, 16)]
        best = jnp.full((16,), -jnp.inf, jnp.float32)
        bidx = jnp.zeros((16,), jnp.int32)
        for l in range(_L):
            tot = plsc.load_gather(
                locfull_v, [jnp.full((16,), l), iota16 + n0])
            cl = jnp.full((16,), cv[l])
            m = jnp.where(cl > 0.0, tot / jnp.maximum(cl, 1.0), 0.0)
            plsc.store_scatter(ratesbuf_v, [iota16 * _L + l], m)
            upd = m > best
            best = jnp.where(upd, m, best)
            bidx = jnp.where(upd, l, bidx)
        abuf_v[...] = bidx
        pltpu.sync_copy(abuf_v, assign_hbm.at[pl.ds(n0, 16)])
        pltpu.sync_copy(ratesbuf_v, rates_hbm.at[pl.ds(n0 * _L, 16 * _L)])


_sc_segment = functools.partial(
    pl.kernel,
    mesh=plsc.VectorSubcoreMesh(core_axis_name="c", subcore_axis_name="s",
                                num_cores=1),
    compiler_params=pltpu.CompilerParams(needs_layout_passes=False),
    out_type=[
        jax.ShapeDtypeStruct((_N,), jnp.int32),
        jax.ShapeDtypeStruct((_N * _L,), jnp.float32),
    ],
    scratch_types=[
        pltpu.VMEM((2, 128, _N), jnp.float32),
        pltpu.VMEM((_RPT,), jnp.int32),
        pltpu.VMEM((2, 128), jnp.int32),
        pltpu.VMEM((16, _N), jnp.float32),
        pltpu.VMEM((16, _N), jnp.float32),
        pltpu.VMEM((1, _N), jnp.float32),
        pltpu.VMEM((_NSUB, _N), jnp.float32),
        pltpu.VMEM((16 * _L,), jnp.float32),
        pltpu.VMEM((16,), jnp.int32),
        pltpu.SemaphoreType.DMA,
        pltpu.SemaphoreType.DMA,
        pltpu.VMEM_SHARED((32, _N), jnp.float32),
        pltpu.VMEM_SHARED((_NSUB, _N), jnp.float32),
    ],
)(_sc_body)


def kernel(spikes, labels):
    B, T, N = spikes.shape
    st = jnp.transpose(spikes, (1, 0, 2))
    ssum = _time_reduce(st)
    assign, rates_flat = _sc_segment(ssum, labels.astype(jnp.int32))
    return assign, rates_flat.reshape(N, _L)
